# Initial kernel scaffold; baseline (speedup 1.0000x reference)
import jax
import jax.numpy as jnp
from jax.experimental import pallas as pl


def kernel(x, edge_attr, params, edge_index, batch_index):
    raise NotImplementedError("write your pallas kernel here")


# speedup vs baseline: 43.7782x; 43.7782x over previous
import functools

import jax
import jax.numpy as jnp
import numpy as np
from jax import lax
from jax.experimental import pallas as pl
from jax.experimental.pallas import tpu as pltpu
from jax.experimental.pallas import tpu_sc as plsc

H = 4
C = 32
D = H * C
G = 128
NC = 2
NS = 16
L = 16
NW = NC * NS

INV_SQRT_C = float(1.0 / np.sqrt(np.float32(C)))


def _proj_body(h_ref, w_ref, b_ref, q_ref, k_ref, v_ref, xr_ref):
    full = jnp.dot(h_ref[...], w_ref[...],
                   preferred_element_type=jnp.float32) + b_ref[...]
    q_ref[...] = full[:, 0:D]
    k_ref[...] = full[:, D:2 * D]
    v_ref[...] = full[:, 2 * D:3 * D]
    xr_ref[...] = full[:, 3 * D:4 * D]


def _proj(h, wcat, bcat):
    n = h.shape[0]
    bn = 1000
    grid = n // bn
    return pl.pallas_call(
        _proj_body,
        grid=(grid,),
        in_specs=[
            pl.BlockSpec((bn, D), lambda i: (i, 0)),
            pl.BlockSpec((D, 4 * D), lambda i: (0, 0)),
            pl.BlockSpec((1, 4 * D), lambda i: (0, 0)),
        ],
        out_specs=[
            pl.BlockSpec((bn, D), lambda i: (i, 0)),
            pl.BlockSpec((bn, D), lambda i: (i, 0)),
            pl.BlockSpec((bn, D), lambda i: (i, 0)),
            pl.BlockSpec((bn, D), lambda i: (i, 0)),
        ],
        out_shape=[
            jax.ShapeDtypeStruct((n, D), jnp.float32),
            jax.ShapeDtypeStruct((n, D), jnp.float32),
            jax.ShapeDtypeStruct((n, D), jnp.float32),
            jax.ShapeDtypeStruct((n, D), jnp.float32),
        ],
    )(h, wcat, bcat)


def _e_body(a_ref, w_ref, o_ref):
    o_ref[...] = jnp.dot(a_ref[...], w_ref[...],
                         preferred_element_type=jnp.float32)


def _e_matmul(edge_attr, we):
    e_num, ed = edge_attr.shape
    rb = 4000
    grid = e_num // rb
    return pl.pallas_call(
        _e_body,
        grid=(grid,),
        in_specs=[
            pl.BlockSpec((rb, ed), lambda i: (i, 0)),
            pl.BlockSpec((ed, D), lambda i: (0, 0)),
        ],
        out_specs=pl.BlockSpec((rb, D), lambda i: (i, 0)),
        out_shape=jax.ShapeDtypeStruct((e_num, D), jnp.float32),
    )(edge_attr, we)


B = 64
GE = 16
NPAD = 10240
DR = NPAD * H // 128


def _sc_edge(q, k, v, e, src, dst):
    n_pad = NPAD
    e_num = src.shape[0]
    nb = e_num // B
    rows_per_tile = n_pad // NS
    mesh = plsc.VectorSubcoreMesh(core_axis_name="c", subcore_axis_name="s",
                                  num_cores=NC, num_subcores=NS)

    @functools.partial(
        pl.kernel,
        out_type=(jax.ShapeDtypeStruct((NC, n_pad, D), jnp.float32),
                  jax.ShapeDtypeStruct((NC, DR, D), jnp.float32)),
        mesh=mesh,
        scratch_types=[
            pltpu.VMEM_SHARED((n_pad, D), jnp.float32),
            pltpu.VMEM_SHARED((DR, D), jnp.float32),
            pltpu.VMEM((1, B), jnp.int32),
            pltpu.VMEM((1, B), jnp.int32),
            pltpu.VMEM((1, B), jnp.int32),
            pltpu.VMEM((B, D), jnp.float32),
            pltpu.VMEM((B, D), jnp.float32),
            pltpu.VMEM((B, D), jnp.float32),
            pltpu.VMEM((B + 1, D), jnp.float32),
            pltpu.SemaphoreType.DMA,
            pltpu.SemaphoreType.DMA,
            pltpu.SemaphoreType.DMA,
            pltpu.SemaphoreType.DMA,
            pltpu.SemaphoreType.DMA,
        ],
    )
    def body(q_hbm, k_hbm, v_hbm, e_hbm, src_hbm, dst_hbm,
             out_hbm, den_hbm,
             acc, dacc, srcv, dstv, dridx, qv, kvx, vvx, ev,
             s1, s2, s3, s4, s5):
        cid = lax.axis_index("c")
        sid = lax.axis_index("s")
        w = sid * NC + cid

        lane = lax.iota(jnp.int32, L)
        zero16 = jnp.zeros((L,), jnp.float32)
        lsplat = jnp.full((L,), L, jnp.int32)
        rot_idx = [lax.rem(lane + sh, lsplat) for sh in (8, 4, 2, 1)]
        gdn = lax.GatherDimensionNumbers(offset_dims=(),
                                         collapsed_slice_dims=(0,),
                                         start_index_map=(0,))

        def _allsum(x):
            for idx in rot_idx:
                x = x + lax.gather(x, idx[:, None], gdn, (1,),
                                   mode=lax.GatherScatterMode.PROMISE_IN_BOUNDS)
            return x

        m_eq0 = lane == 0
        m_eq2 = lane == 2
        m_lt2 = lane < 2
        m_lt4 = lane < H

        @pl.loop(0, B)
        def _(r):
            for ccol in range(D // L):
                qv[r, pl.ds(ccol * L, L)] = zero16

        base_row = sid * rows_per_tile
        for t in range(rows_per_tile // B):
            pltpu.sync_copy(qv, acc.at[pl.ds(base_row + t * B, B)])
        @pl.when(sid < DR // 64)
        def _():
            pltpu.sync_copy(qv.at[pl.ds(0, 64)],
                            dacc.at[pl.ds(sid * 64, 64)])
        plsc.subcore_barrier()

        nblk = (nb - w + NW - 1) // NW

        @pl.loop(0, nblk)
        def _(t):
            base = (t * NW + w) * B
            pltpu.sync_copy(src_hbm.at[pl.ds(base, B)], srcv.at[0])
            pltpu.sync_copy(dst_hbm.at[pl.ds(base, B)], dstv.at[0])
            d1 = pltpu.async_copy(k_hbm.at[srcv.at[0]], kvx, s1)
            d2 = pltpu.async_copy(v_hbm.at[srcv.at[0]], vvx, s2)
            d3 = pltpu.async_copy(q_hbm.at[dstv.at[0]], qv, s3)
            d4 = pltpu.async_copy(e_hbm.at[pl.ds(base, B)], ev.at[pl.ds(0, B)],
                                  s4)
            for ccol in range(B // L):
                dd = dstv[0, pl.ds(ccol * L, L)]
                dridx[0, pl.ds(ccol * L, L)] = lax.shift_right_logical(dd, 5)
            d1.wait()
            d2.wait()
            d3.wait()
            d4.wait()

            @pl.loop(0, B // GE)
            def _(g):
                dsts16 = dstv[0, pl.ds(g * GE, GE)]
                for j in range(GE):
                    i = g * GE + j
                    prods = []
                    ees = []
                    for cc in range(D // L):
                        qq = qv[i, pl.ds(cc * L, L)]
                        kk = kvx[i, pl.ds(cc * L, L)]
                        ee = ev[i, pl.ds(cc * L, L)]
                        ees.append(ee)
                        prods.append(qq * (kk + ee))
                    exs = []
                    for h in range(H):
                        sv = _allsum(prods[2 * h] + prods[2 * h + 1])
                        exs.append(jnp.exp(sv * INV_SQRT_C))
                    for h in range(H):
                        for cc in (2 * h, 2 * h + 1):
                            vvv = vvx[i, pl.ds(cc * L, L)]
                            qv[i, pl.ds(cc * L, L)] = (vvv + ees[cc]) * exs[h]
                    d_i = dsts16[j]
                    for cc in range(D // L):
                        ev[i, pl.ds(cc * L, L)] = zero16
                    dv = jnp.where(m_lt2,
                                   jnp.where(m_eq0, exs[0], exs[1]),
                                   jnp.where(m_eq2, exs[2], exs[3]))
                    dv = jnp.where(m_lt4, dv, 0.0)
                    colo = lax.shift_left(lax.bitwise_and(d_i, 31), 2)
                    p = lax.bitwise_and(colo, 12)
                    base16 = colo - p
                    ridx = lax.bitwise_and(lane + (16 - p), 15)
                    dvs = lax.gather(dv, ridx[:, None], gdn, (1,),
                                     mode=lax.GatherScatterMode.PROMISE_IN_BOUNDS)
                    ev[i, pl.ds(base16, L)] = dvs

            d5 = pltpu.async_copy(qv, acc.at[dstv.at[0]], s5, add=True)
            d6 = pltpu.async_copy(ev.at[pl.ds(0, B)], dacc.at[dridx.at[0]],
                                  s4, add=True)
            d5.wait()
            d6.wait()

        plsc.subcore_barrier()

        pltpu.sync_copy(acc.at[pl.ds(base_row, rows_per_tile)],
                        out_hbm.at[cid, pl.ds(base_row, rows_per_tile)])

        @pl.when(sid < DR // 64)
        def _():
            pltpu.sync_copy(dacc.at[pl.ds(sid * 64, 64)],
                            den_hbm.at[cid, pl.ds(sid * 64, 64)])

    return body(q, k, v, e, src, dst)


def _gate_body(p_ref, d_ref, xr_ref, wba_ref, wbb_ref, o_ref):
    outp = p_ref[0] + p_ref[1]
    bn = outp.shape[0]
    den = d_ref[0] + d_ref[1]
    div = jnp.concatenate(
        [jnp.broadcast_to(den[:, h:h + 1], (bn, C)) for h in range(H)], axis=1)
    out = outp / (div + 1e-16)
    xr = xr_ref[...]
    lin = (jnp.sum(xr * wba_ref[...], axis=1, keepdims=True)
           + jnp.sum(out * wbb_ref[...], axis=1, keepdims=True))
    beta = jax.nn.sigmoid(lin)
    o_ref[...] = beta * xr + (1.0 - beta) * out


def _gate(parts, den3, xr, wba, wbb):
    n = xr.shape[0]
    bn = 1000
    grid = n // bn
    return pl.pallas_call(
        _gate_body,
        grid=(grid,),
        in_specs=[
            pl.BlockSpec((NC, bn, D), lambda i: (0, i, 0)),
            pl.BlockSpec((NC, bn, H), lambda i: (0, i, 0)),
            pl.BlockSpec((bn, D), lambda i: (i, 0)),
            pl.BlockSpec((1, D), lambda i: (0, 0)),
            pl.BlockSpec((1, D), lambda i: (0, 0)),
        ],
        out_specs=pl.BlockSpec((bn, D), lambda i: (i, 0)),
        out_shape=jax.ShapeDtypeStruct((n, D), jnp.float32),
    )(parts, den3, xr, wba, wbb)


def _pool_body(h_ref, b_ref, w1_ref, b1_ref, w2_ref, b2_ref, o_ref,
               acc_s, cnt_s):
    i = pl.program_id(0)

    @pl.when(i == 0)
    def _():
        acc_s[...] = jnp.zeros_like(acc_s)
        cnt_s[...] = jnp.zeros_like(cnt_s)

    gi = lax.broadcasted_iota(jnp.int32, (G, 1), 0).astype(jnp.float32)
    bvals = b_ref[0]
    mask = (gi == bvals).astype(jnp.float32)
    acc_s[...] += jnp.dot(mask, h_ref[...], preferred_element_type=jnp.float32)
    cnt_s[...] += jnp.sum(mask, axis=1, keepdims=True)

    @pl.when(i == pl.num_programs(0) - 1)
    def _():
        pooled = acc_s[...] / jnp.maximum(cnt_s[...], 1.0)
        h1 = jnp.maximum(
            jnp.dot(pooled, w1_ref[...], preferred_element_type=jnp.float32)
            + b1_ref[...], 0.0)
        o_ref[...] = (jnp.dot(h1, w2_ref[...],
                              preferred_element_type=jnp.float32)
                      + b2_ref[...])


def _pool(h, bidx3, w1, b1, w2, b2):
    n = h.shape[0]
    bn = 1000
    grid = n // bn
    return pl.pallas_call(
        _pool_body,
        grid=(grid,),
        in_specs=[
            pl.BlockSpec((bn, D), lambda i: (i, 0)),
            pl.BlockSpec((1, 1, bn), lambda i: (i, 0, 0)),
            pl.BlockSpec((D, 32), lambda i: (0, 0)),
            pl.BlockSpec((1, 32), lambda i: (0, 0)),
            pl.BlockSpec((32, 10), lambda i: (0, 0)),
            pl.BlockSpec((1, 10), lambda i: (0, 0)),
        ],
        out_specs=pl.BlockSpec((G, 10), lambda i: (0, 0)),
        out_shape=jax.ShapeDtypeStruct((G, 10), jnp.float32),
        scratch_shapes=[
            pltpu.VMEM((G, D), jnp.float32),
            pltpu.VMEM((G, 1), jnp.float32),
        ],
    )(h, bidx3, w1, b1, w2, b2)


def kernel(x, edge_attr, params, edge_index, batch_index):
    src = edge_index[0]
    dst = edge_index[1]
    n = x.shape[0]

    h = x
    for lp in params["layers"]:
        wcat = jnp.concatenate(
            [lp["Wq"], lp["Wk"], lp["Wv"], lp["Ws"]], axis=1)
        bcat = jnp.concatenate(
            [lp["bq"], lp["bk"], lp["bv"], lp["bs"]]).reshape(1, 4 * D)
        wb = lp["Wb"][:, 0]
        wba = (wb[0:D] + wb[2 * D:3 * D]).reshape(1, D)
        wbb = (wb[D:2 * D] - wb[2 * D:3 * D]).reshape(1, D)

        q, kk, vv, xr = _proj(h, wcat, bcat)
        e = _e_matmul(edge_attr, lp["We"])
        parts, dens = _sc_edge(q, kk, vv, e, src, dst)
        den3 = dens.reshape(NC, DR * D // H, H)
        h = _gate(parts, den3, xr, wba, wbb)

    bn = 1000
    bidx3 = batch_index.astype(jnp.float32).reshape(n // bn, 1, bn)
    return _pool(h, bidx3, params["ro_W1"], params["ro_b1"].reshape(1, 32),
                 params["ro_W2"], params["ro_b2"].reshape(1, 10))
